# baseline (speedup 1.0000x reference)
import functools
import jax
import jax.numpy as jnp
from jax import lax
from jax.experimental import pallas as pl
from jax.experimental.pallas import tpu as pltpu
from jax.experimental.pallas import tpu_sc as plsc

NN = 10000
NP = 10240
D = 128
KHOPS = 3
NC = 2
NS = 16
RPT = NP // NS
EB = 112
CPB = 8
CHUNK = CPB * EB
CPT = 12
EP = CHUNK * CPT * NC * NS
NCHUNKS = EP // CHUNK
BPT = CPT * CPB

_mesh = plsc.VectorSubcoreMesh(core_axis_name="c", subcore_axis_name="s",
                               num_cores=NC, num_subcores=NS)


@functools.partial(
    pl.kernel,
    out_type=jax.ShapeDtypeStruct((NC, NP), jnp.float32),
    mesh=_mesh,
    scratch_types=[
        pltpu.VMEM((CPB, EB), jnp.int32),
        pltpu.VMEM((EB,), jnp.float32),
        pltpu.VMEM_SHARED((NP,), jnp.float32),
        pltpu.SemaphoreType.DMA,
    ],
)
def _deg_kernel(col2d, zeros1, deg_out, cidx, ones_v, acc, sem):
    del sem
    cid = lax.axis_index("c")
    sid = lax.axis_index("s")
    for j in range(EB // 16):
        ones_v[pl.ds(j * 16, 16)] = jnp.ones((16,), jnp.float32)
    pltpu.sync_copy(zeros1.at[pl.ds(sid * RPT, RPT)],
                    acc.at[pl.ds(sid * RPT, RPT)])
    plsc.subcore_barrier()
    wid = cid * NS + sid

    def body(ch, carry):
        g = wid * CPT + ch
        pltpu.sync_copy(col2d.at[pl.ds(g * CPB, CPB)], cidx)
        for j in range(CPB):
            pltpu.sync_copy(ones_v, acc.at[cidx.at[j]], add=True)
        return carry

    lax.fori_loop(0, CPT, body, 0)
    plsc.subcore_barrier()
    pltpu.sync_copy(acc.at[pl.ds(sid * RPT, RPT)],
                    deg_out.at[cid, pl.ds(sid * RPT, RPT)])


@functools.partial(
    pl.kernel,
    out_type=jax.ShapeDtypeStruct((NC, NP, D), jnp.float32),
    mesh=_mesh,
    scratch_types=[
        pltpu.VMEM((2, CPB, EB), jnp.int32),
        pltpu.VMEM((2, CPB, EB), jnp.int32),
        pltpu.VMEM((3, EB, D), jnp.float32),
        pltpu.VMEM_SHARED((NP, D), jnp.float32),
        pltpu.SemaphoreType.DMA,
        pltpu.SemaphoreType.DMA((3,)),
        pltpu.SemaphoreType.DMA,
    ],
)
def _prop_kernel(u, row2d, col2d, zeros2, s_out,
                 ribuf, cibuf, gbuf, acc, sem_g, sem_s, sem_i):
    cid = lax.axis_index("c")
    sid = lax.axis_index("s")
    wid = cid * NS + sid
    pltpu.async_copy(row2d.at[pl.ds(wid * CPT * CPB, CPB)], ribuf.at[0],
                     sem_i)
    pltpu.async_copy(col2d.at[pl.ds(wid * CPT * CPB, CPB)], cibuf.at[0],
                     sem_i)
    pltpu.async_copy(zeros2.at[pl.ds(sid * RPT, RPT)],
                     acc.at[pl.ds(sid * RPT, RPT)], sem_g)
    pltpu.make_async_copy(row2d.at[pl.ds(0, CPB)], ribuf.at[0], sem_i).wait()
    pltpu.make_async_copy(col2d.at[pl.ds(0, CPB)], cibuf.at[0], sem_i).wait()
    pltpu.make_async_copy(zeros2.at[pl.ds(0, RPT)],
                          acc.at[pl.ds(0, RPT)], sem_g).wait()
    plsc.subcore_barrier()

    pltpu.async_copy(u.at[ribuf.at[0, 0]], gbuf.at[0], sem_g)
    pltpu.async_copy(u.at[ribuf.at[0, 1]], gbuf.at[1], sem_g)

    def super_body(sc, carry):
        for j in range(3):
            c = 3 * sc + j
            pc = lax.rem(c, 2)
            qc = 1 - pc
            for b in range(CPB):
                s0 = (2 * j + b) % 3
                sm = (2 * j + b + 2) % 3
                pltpu.make_async_copy(u.at[ribuf.at[pc, b]], gbuf.at[s0],
                                      sem_g).wait()
                pltpu.async_copy(gbuf.at[s0], acc.at[cibuf.at[pc, b]],
                                 sem_s.at[s0], add=True)

                if b > 0:
                    pltpu.make_async_copy(gbuf.at[sm],
                                          acc.at[cibuf.at[pc, b - 1]],
                                          sem_s.at[sm]).wait()
                else:
                    def drain_prev():
                        pltpu.make_async_copy(gbuf.at[sm],
                                              acc.at[cibuf.at[qc, CPB - 1]],
                                              sem_s.at[sm]).wait()

                    def prefetch():
                        g1 = (wid * CPT + c + 1) * CPB
                        pltpu.async_copy(row2d.at[pl.ds(g1, CPB)],
                                         ribuf.at[qc], sem_i)
                        pltpu.async_copy(col2d.at[pl.ds(g1, CPB)],
                                         cibuf.at[qc], sem_i)

                    if j == 1:
                        drain_prev()
                        prefetch()
                    elif j == 2:
                        drain_prev()

                        @pl.when(c < CPT - 1)
                        def _():
                            prefetch()
                    else:
                        @pl.when(c > 0)
                        def _():
                            drain_prev()

                        @pl.when(c < CPT - 1)
                        def _():
                            prefetch()

                if b + 2 < CPB:
                    pltpu.async_copy(u.at[ribuf.at[pc, b + 2]],
                                     gbuf.at[sm], sem_g)
                else:
                    def fire_next():
                        if b == CPB - 2:
                            pltpu.make_async_copy(row2d.at[pl.ds(0, CPB)],
                                                  ribuf.at[qc], sem_i).wait()
                            pltpu.make_async_copy(col2d.at[pl.ds(0, CPB)],
                                                  cibuf.at[qc], sem_i).wait()
                        pltpu.async_copy(u.at[ribuf.at[qc, b + 2 - CPB]],
                                         gbuf.at[sm], sem_g)

                    if j < 2:
                        fire_next()
                    else:
                        @pl.when(c < CPT - 1)
                        def _():
                            fire_next()
        return carry

    lax.fori_loop(0, CPT // 3, super_body, 0)
    s_fin = (2 * 2 + (CPB - 1)) % 3
    pltpu.make_async_copy(gbuf.at[s_fin],
                          acc.at[cibuf.at[(CPT - 1) % 2, CPB - 1]],
                          sem_s.at[s_fin]).wait()
    plsc.subcore_barrier()
    pltpu.sync_copy(acc.at[pl.ds(sid * RPT, RPT)],
                    s_out.at[cid, pl.ds(sid * RPT, RPT)])


_HI = lax.Precision.HIGHEST
BR = 1024
NB = NP // BR


def _rowmask(i, val, alt):
    rowv = i * BR + lax.broadcasted_iota(jnp.int32, (BR, 1), 0)
    return jnp.where(rowv < NN, val, alt)


def _prep_body(dega, degb, x, dis, dis2, u):
    d = dega[...] + degb[...]
    r = jnp.where(d > 0, lax.rsqrt(jnp.maximum(d, 1e-12)), 0.0)
    dis[...] = r
    dis2[...] = r * r
    u[...] = _rowmask(pl.program_id(0), r * x[...], 0.0)


def _prep(deg2, x):
    dega = deg2[0].reshape(NP, 1)
    degb = deg2[1].reshape(NP, 1)
    return pl.pallas_call(
        _prep_body,
        grid=(NB,),
        in_specs=[
            pl.BlockSpec((BR, 1), lambda i: (i, 0)),
            pl.BlockSpec((BR, 1), lambda i: (i, 0)),
            pl.BlockSpec((BR, D), lambda i: (i, 0)),
        ],
        out_specs=[
            pl.BlockSpec((BR, 1), lambda i: (i, 0)),
            pl.BlockSpec((BR, 1), lambda i: (i, 0)),
            pl.BlockSpec((BR, D), lambda i: (i, 0)),
        ],
        out_shape=[
            jax.ShapeDtypeStruct((NP, 1), jnp.float32),
            jax.ShapeDtypeStruct((NP, 1), jnp.float32),
            jax.ShapeDtypeStruct((NP, D), jnp.float32),
        ],
    )(dega, degb, x)


def _uscale_body(s, dis2, u):
    sb = s[...]
    u[...] = dis2[...] * (sb[0] + sb[1])


def _uscale(s, dis2):
    return pl.pallas_call(
        _uscale_body,
        grid=(NB,),
        in_specs=[
            pl.BlockSpec((NC, BR, D), lambda i: (0, i, 0)),
            pl.BlockSpec((BR, 1), lambda i: (i, 0)),
        ],
        out_specs=pl.BlockSpec((BR, D), lambda i: (i, 0)),
        out_shape=jax.ShapeDtypeStruct((NP, D), jnp.float32),
    )(s, dis2)


def _matmul_part_body(x, s1, s2, dis, w, out):
    dv = dis[...]
    acc = jnp.dot(x[...], w[0], precision=_HI)
    for k, s in enumerate((s1, s2)):
        sb = s[...]
        acc = acc + jnp.dot(dv * (sb[0] + sb[1]), w[k + 1], precision=_HI)
    out[...] = acc


def _matmul_part(x, s1, s2, dis, w):
    sspec = pl.BlockSpec((NC, BR, D), lambda i: (0, i, 0))
    return pl.pallas_call(
        _matmul_part_body,
        grid=(NB,),
        in_specs=[
            pl.BlockSpec((BR, D), lambda i: (i, 0)),
            sspec, sspec,
            pl.BlockSpec((BR, 1), lambda i: (i, 0)),
            pl.BlockSpec((KHOPS + 1, D, D), lambda i: (0, 0, 0)),
        ],
        out_specs=pl.BlockSpec((BR, D), lambda i: (i, 0)),
        out_shape=jax.ShapeDtypeStruct((NP, D), jnp.float32),
    )(x, s1, s2, dis, w)


def _matmul_fin_body(relu_u, part, s3, dis, w, b, out, u=None):
    dv = dis[...]
    sb = s3[...]
    acc = part[...] + jnp.dot(dv * (sb[0] + sb[1]), w[KHOPS],
                              precision=_HI)
    acc = acc + b[...]
    if relu_u:
        acc = jnp.maximum(acc, 0.0)
        u[...] = _rowmask(pl.program_id(0), dv * acc, 0.0)
    out[...] = acc


def _matmul_fin(part, s3, dis, w, b, relu_u):
    out_shape = [jax.ShapeDtypeStruct((NN, D), jnp.float32)]
    out_specs = [pl.BlockSpec((BR, D), lambda i: (i, 0))]
    if relu_u:
        out_shape.append(jax.ShapeDtypeStruct((NP, D), jnp.float32))
        out_specs.append(pl.BlockSpec((BR, D), lambda i: (i, 0)))
    return pl.pallas_call(
        functools.partial(_matmul_fin_body, relu_u),
        grid=(NB,),
        in_specs=[
            pl.BlockSpec((BR, D), lambda i: (i, 0)),
            pl.BlockSpec((NC, BR, D), lambda i: (0, i, 0)),
            pl.BlockSpec((BR, 1), lambda i: (i, 0)),
            pl.BlockSpec((KHOPS + 1, D, D), lambda i: (0, 0, 0)),
            pl.BlockSpec((1, D), lambda i: (0, 0)),
        ],
        out_specs=out_specs,
        out_shape=out_shape,
    )(part, s3, dis, w, b)


def kernel(x, edge_index, W1, b1, W2, b2):
    npad = EP - edge_index.shape[1]
    padv = NN + (jnp.arange(npad, dtype=jnp.int32) % (NP - NN))
    pad = jnp.stack([padv, padv])
    ei = jnp.concatenate([edge_index.astype(jnp.int32), pad], axis=1)
    row2d = ei[0].reshape(EP // EB, EB)
    col2d = ei[1].reshape(EP // EB, EB)
    zeros1 = jnp.zeros((NP,), jnp.float32)
    zeros2 = jnp.zeros((NP, D), jnp.float32)
    b1r = b1.reshape(1, D)
    b2r = b2.reshape(1, D)

    deg2 = _deg_kernel(col2d, zeros1)
    dis, dis2, u = _prep(deg2, x)

    def layer(xin, u, w, br, relu_u):
        s1 = _prop_kernel(u, row2d, col2d, zeros2)
        u1 = _uscale(s1, dis2)
        s2 = _prop_kernel(u1, row2d, col2d, zeros2)
        u2 = _uscale(s2, dis2)
        s3 = _prop_kernel(u2, row2d, col2d, zeros2)
        part = _matmul_part(xin, s1, s2, dis, w)
        return _matmul_fin(part, s3, dis, w, br, relu_u)

    x2, u2 = layer(x, u, W1, b1r, True)
    (out,) = layer(x2, u2, W2, b2r, False)
    return out.reshape(-1)


# speedup vs baseline: 1.0005x; 1.0005x over previous
import functools
import jax
import jax.numpy as jnp
from jax import lax
from jax.experimental import pallas as pl
from jax.experimental.pallas import tpu as pltpu
from jax.experimental.pallas import tpu_sc as plsc

NN = 10000
NP = 10240
D = 128
KHOPS = 3
NC = 2
NS = 16
RPT = NP // NS
EB = 112
CPB = 8
CHUNK = CPB * EB
CPT = 12
EP = CHUNK * CPT * NC * NS
NCHUNKS = EP // CHUNK
BPT = CPT * CPB

_mesh = plsc.VectorSubcoreMesh(core_axis_name="c", subcore_axis_name="s",
                               num_cores=NC, num_subcores=NS)


DEB = 125
DCPB = 8
DCPT = 10


@functools.partial(
    pl.kernel,
    out_type=jax.ShapeDtypeStruct((NC, NP), jnp.float32),
    mesh=_mesh,
    scratch_types=[
        pltpu.VMEM((DCPB, DEB), jnp.int32),
        pltpu.VMEM((128,), jnp.float32),
        pltpu.VMEM_SHARED((NP,), jnp.float32),
        pltpu.SemaphoreType.DMA,
    ],
)
def _deg_kernel(col2d, zeros1, deg_out, cidx, ones_v, acc, sem):
    del sem
    cid = lax.axis_index("c")
    sid = lax.axis_index("s")
    for j in range(8):
        ones_v[pl.ds(j * 16, 16)] = jnp.ones((16,), jnp.float32)
    pltpu.sync_copy(zeros1.at[pl.ds(sid * RPT, RPT)],
                    acc.at[pl.ds(sid * RPT, RPT)])
    plsc.subcore_barrier()
    wid = cid * NS + sid

    def body(ch, carry):
        g = wid * DCPT + ch
        pltpu.sync_copy(col2d.at[pl.ds(g * DCPB, DCPB)], cidx)
        for j in range(DCPB):
            pltpu.sync_copy(ones_v.at[pl.ds(0, DEB)],
                            acc.at[cidx.at[j]], add=True)
        return carry

    lax.fori_loop(0, DCPT, body, 0)
    plsc.subcore_barrier()
    pltpu.sync_copy(acc.at[pl.ds(sid * RPT, RPT)],
                    deg_out.at[cid, pl.ds(sid * RPT, RPT)])


@functools.partial(
    pl.kernel,
    out_type=jax.ShapeDtypeStruct((NC, NP, D), jnp.float32),
    mesh=_mesh,
    scratch_types=[
        pltpu.VMEM((2, CPB, EB), jnp.int32),
        pltpu.VMEM((2, CPB, EB), jnp.int32),
        pltpu.VMEM((3, EB, D), jnp.float32),
        pltpu.VMEM_SHARED((NP, D), jnp.float32),
        pltpu.SemaphoreType.DMA,
        pltpu.SemaphoreType.DMA((3,)),
        pltpu.SemaphoreType.DMA,
    ],
)
def _prop_kernel(u, row2d, col2d, zeros2, s_out,
                 ribuf, cibuf, gbuf, acc, sem_g, sem_s, sem_i):
    cid = lax.axis_index("c")
    sid = lax.axis_index("s")
    wid = cid * NS + sid
    pltpu.async_copy(row2d.at[pl.ds(wid * CPT * CPB, CPB)], ribuf.at[0],
                     sem_i)
    pltpu.async_copy(col2d.at[pl.ds(wid * CPT * CPB, CPB)], cibuf.at[0],
                     sem_i)
    pltpu.async_copy(zeros2.at[pl.ds(sid * RPT, RPT)],
                     acc.at[pl.ds(sid * RPT, RPT)], sem_g)
    pltpu.make_async_copy(row2d.at[pl.ds(0, CPB)], ribuf.at[0], sem_i).wait()
    pltpu.make_async_copy(col2d.at[pl.ds(0, CPB)], cibuf.at[0], sem_i).wait()
    pltpu.make_async_copy(zeros2.at[pl.ds(0, RPT)],
                          acc.at[pl.ds(0, RPT)], sem_g).wait()
    plsc.subcore_barrier()

    pltpu.async_copy(u.at[ribuf.at[0, 0]], gbuf.at[0], sem_g)
    pltpu.async_copy(u.at[ribuf.at[0, 1]], gbuf.at[1], sem_g)

    def super_body(sc, carry):
        for j in range(3):
            c = 3 * sc + j
            pc = lax.rem(c, 2)
            qc = 1 - pc
            for b in range(CPB):
                s0 = (2 * j + b) % 3
                sm = (2 * j + b + 2) % 3
                pltpu.make_async_copy(u.at[ribuf.at[pc, b]], gbuf.at[s0],
                                      sem_g).wait()
                pltpu.async_copy(gbuf.at[s0], acc.at[cibuf.at[pc, b]],
                                 sem_s.at[s0], add=True)

                if b > 0:
                    pltpu.make_async_copy(gbuf.at[sm],
                                          acc.at[cibuf.at[pc, b - 1]],
                                          sem_s.at[sm]).wait()
                else:
                    def drain_prev():
                        pltpu.make_async_copy(gbuf.at[sm],
                                              acc.at[cibuf.at[qc, CPB - 1]],
                                              sem_s.at[sm]).wait()

                    def prefetch():
                        g1 = (wid * CPT + c + 1) * CPB
                        pltpu.async_copy(row2d.at[pl.ds(g1, CPB)],
                                         ribuf.at[qc], sem_i)
                        pltpu.async_copy(col2d.at[pl.ds(g1, CPB)],
                                         cibuf.at[qc], sem_i)

                    if j == 1:
                        drain_prev()
                        prefetch()
                    elif j == 2:
                        drain_prev()

                        @pl.when(c < CPT - 1)
                        def _():
                            prefetch()
                    else:
                        @pl.when(c > 0)
                        def _():
                            drain_prev()

                        @pl.when(c < CPT - 1)
                        def _():
                            prefetch()

                if b + 2 < CPB:
                    pltpu.async_copy(u.at[ribuf.at[pc, b + 2]],
                                     gbuf.at[sm], sem_g)
                else:
                    def fire_next():
                        if b == CPB - 2:
                            pltpu.make_async_copy(row2d.at[pl.ds(0, CPB)],
                                                  ribuf.at[qc], sem_i).wait()
                            pltpu.make_async_copy(col2d.at[pl.ds(0, CPB)],
                                                  cibuf.at[qc], sem_i).wait()
                        pltpu.async_copy(u.at[ribuf.at[qc, b + 2 - CPB]],
                                         gbuf.at[sm], sem_g)

                    if j < 2:
                        fire_next()
                    else:
                        @pl.when(c < CPT - 1)
                        def _():
                            fire_next()
        return carry

    lax.fori_loop(0, CPT // 3, super_body, 0)
    s_fin = (2 * 2 + (CPB - 1)) % 3
    pltpu.make_async_copy(gbuf.at[s_fin],
                          acc.at[cibuf.at[(CPT - 1) % 2, CPB - 1]],
                          sem_s.at[s_fin]).wait()
    plsc.subcore_barrier()
    pltpu.sync_copy(acc.at[pl.ds(sid * RPT, RPT)],
                    s_out.at[cid, pl.ds(sid * RPT, RPT)])


_HI = lax.Precision.HIGHEST
BR = 1024
NB = NP // BR


def _rowmask(i, val, alt):
    rowv = i * BR + lax.broadcasted_iota(jnp.int32, (BR, 1), 0)
    return jnp.where(rowv < NN, val, alt)


def _prep_body(dega, degb, x, dis, dis2, u):
    d = dega[...] + degb[...]
    r = jnp.where(d > 0, lax.rsqrt(jnp.maximum(d, 1e-12)), 0.0)
    dis[...] = r
    dis2[...] = r * r
    u[...] = _rowmask(pl.program_id(0), r * x[...], 0.0)


def _prep(deg2, x):
    dega = deg2[0].reshape(NP, 1)
    degb = deg2[1].reshape(NP, 1)
    return pl.pallas_call(
        _prep_body,
        grid=(NB,),
        in_specs=[
            pl.BlockSpec((BR, 1), lambda i: (i, 0)),
            pl.BlockSpec((BR, 1), lambda i: (i, 0)),
            pl.BlockSpec((BR, D), lambda i: (i, 0)),
        ],
        out_specs=[
            pl.BlockSpec((BR, 1), lambda i: (i, 0)),
            pl.BlockSpec((BR, 1), lambda i: (i, 0)),
            pl.BlockSpec((BR, D), lambda i: (i, 0)),
        ],
        out_shape=[
            jax.ShapeDtypeStruct((NP, 1), jnp.float32),
            jax.ShapeDtypeStruct((NP, 1), jnp.float32),
            jax.ShapeDtypeStruct((NP, D), jnp.float32),
        ],
    )(dega, degb, x)


def _uscale_body(s, dis2, u):
    sb = s[...]
    u[...] = dis2[...] * (sb[0] + sb[1])


def _uscale(s, dis2):
    return pl.pallas_call(
        _uscale_body,
        grid=(NB,),
        in_specs=[
            pl.BlockSpec((NC, BR, D), lambda i: (0, i, 0)),
            pl.BlockSpec((BR, 1), lambda i: (i, 0)),
        ],
        out_specs=pl.BlockSpec((BR, D), lambda i: (i, 0)),
        out_shape=jax.ShapeDtypeStruct((NP, D), jnp.float32),
    )(s, dis2)


def _matmul_part_body(x, s1, s2, dis, w, out):
    dv = dis[...]
    acc = jnp.dot(x[...], w[0], precision=_HI)
    for k, s in enumerate((s1, s2)):
        sb = s[...]
        acc = acc + jnp.dot(dv * (sb[0] + sb[1]), w[k + 1], precision=_HI)
    out[...] = acc


def _matmul_part(x, s1, s2, dis, w):
    sspec = pl.BlockSpec((NC, BR, D), lambda i: (0, i, 0))
    return pl.pallas_call(
        _matmul_part_body,
        grid=(NB,),
        in_specs=[
            pl.BlockSpec((BR, D), lambda i: (i, 0)),
            sspec, sspec,
            pl.BlockSpec((BR, 1), lambda i: (i, 0)),
            pl.BlockSpec((KHOPS + 1, D, D), lambda i: (0, 0, 0)),
        ],
        out_specs=pl.BlockSpec((BR, D), lambda i: (i, 0)),
        out_shape=jax.ShapeDtypeStruct((NP, D), jnp.float32),
    )(x, s1, s2, dis, w)


def _matmul_fin_body(relu_u, part, s3, dis, w, b, out, u=None):
    dv = dis[...]
    sb = s3[...]
    acc = part[...] + jnp.dot(dv * (sb[0] + sb[1]), w[KHOPS],
                              precision=_HI)
    acc = acc + b[...]
    if relu_u:
        acc = jnp.maximum(acc, 0.0)
        u[...] = _rowmask(pl.program_id(0), dv * acc, 0.0)
    out[...] = acc


def _matmul_fin(part, s3, dis, w, b, relu_u):
    out_shape = [jax.ShapeDtypeStruct((NN, D), jnp.float32)]
    out_specs = [pl.BlockSpec((BR, D), lambda i: (i, 0))]
    if relu_u:
        out_shape.append(jax.ShapeDtypeStruct((NP, D), jnp.float32))
        out_specs.append(pl.BlockSpec((BR, D), lambda i: (i, 0)))
    return pl.pallas_call(
        functools.partial(_matmul_fin_body, relu_u),
        grid=(NB,),
        in_specs=[
            pl.BlockSpec((BR, D), lambda i: (i, 0)),
            pl.BlockSpec((NC, BR, D), lambda i: (0, i, 0)),
            pl.BlockSpec((BR, 1), lambda i: (i, 0)),
            pl.BlockSpec((KHOPS + 1, D, D), lambda i: (0, 0, 0)),
            pl.BlockSpec((1, D), lambda i: (0, 0)),
        ],
        out_specs=out_specs,
        out_shape=out_shape,
    )(part, s3, dis, w, b)


def kernel(x, edge_index, W1, b1, W2, b2):
    E0 = edge_index.shape[1]
    npad = EP - E0
    padv = NN + (jnp.arange(npad, dtype=jnp.int32) % (NP - NN))
    pad = jnp.stack([padv, padv])
    ei = jnp.concatenate([edge_index.astype(jnp.int32), pad], axis=1)
    row2d = ei[0].reshape(EP // EB, EB)
    col2d = ei[1].reshape(EP // EB, EB)
    zeros1 = jnp.zeros((NP,), jnp.float32)
    zeros2 = jnp.zeros((NP, D), jnp.float32)
    b1r = b1.reshape(1, D)
    b2r = b2.reshape(1, D)

    col_deg = edge_index.astype(jnp.int32)[1].reshape(E0 // DEB, DEB)
    deg2 = _deg_kernel(col_deg, zeros1)
    dis, dis2, u = _prep(deg2, x)

    def layer(xin, u, w, br, relu_u):
        s1 = _prop_kernel(u, row2d, col2d, zeros2)
        u1 = _uscale(s1, dis2)
        s2 = _prop_kernel(u1, row2d, col2d, zeros2)
        u2 = _uscale(s2, dis2)
        s3 = _prop_kernel(u2, row2d, col2d, zeros2)
        part = _matmul_part(xin, s1, s2, dis, w)
        return _matmul_fin(part, s3, dis, w, br, relu_u)

    x2, u2 = layer(x, u, W1, b1r, True)
    (out,) = layer(x2, u2, W2, b2r, False)
    return out.reshape(-1)
